# baseline (speedup 1.0000x reference)
import jax
import jax.numpy as jnp
from jax import lax
from jax.experimental import pallas as pl

N, S, K = 4096, 200, 100
B_BLK = 64


def _body(in_ref, out_ref):
    ids = in_ref[...]
    iota = lax.broadcasted_iota(jnp.int32, (B_BLK, S, K), 2)
    out_ref[...] = (ids[:, :, None] == iota).astype(jnp.float32)


def kernel(inputs):
    return pl.pallas_call(
        _body,
        grid=(N // B_BLK,),
        in_specs=[pl.BlockSpec((B_BLK, S), lambda i: (i, 0))],
        out_specs=pl.BlockSpec((B_BLK, S, K), lambda i: (i, 0, 0)),
        out_shape=jax.ShapeDtypeStruct((N, S, K), jnp.float32),
    )(inputs)
